# baseline (speedup 1.0000x reference)
import functools

import jax
import jax.numpy as jnp
from jax import lax
from jax.experimental import pallas as pl
from jax.experimental.pallas import tpu as pltpu
from jax.experimental.pallas import tpu_sc as plsc

_N_NODES = 10000
_N_EDGES = 320000
_L = 16
_NS = 16
_NC = 1
_NW = _NC * _NS
_HIST_PAD = 20480
_OUT_OFF = 10240
_SLICE = _HIST_PAD // _NS
_E_HIST = _N_EDGES // _NS
_E_GATH = _N_EDGES // _NW

_mesh = plsc.VectorSubcoreMesh(core_axis_name="c", subcore_axis_name="s",
                               num_cores=_NC)


@functools.partial(
    pl.kernel,
    mesh=_mesh,
    out_type=jax.ShapeDtypeStruct((_N_EDGES,), jnp.float32),
    scratch_types=[
        pltpu.VMEM((_HIST_PAD,), jnp.float32),
        pltpu.VMEM((_E_HIST,), jnp.int32),
        pltpu.VMEM((_E_GATH,), jnp.int32),
        pltpu.VMEM((_E_GATH,), jnp.int32),
        pltpu.VMEM((_E_GATH,), jnp.float32),
        pltpu.VMEM((_SLICE,), jnp.float32),
        pltpu.VMEM((_SLICE,), jnp.float32),
        pltpu.VMEM_SHARED((_NS * _HIST_PAD,), jnp.float32),
    ],
    compiler_params=pltpu.CompilerParams(needs_layout_passes=False),
)
def _sds_kernel(head_hbm, tail_hbm, psrc_hbm, pdst_hbm, out_hbm,
                hist, idx_buf, tail_buf, head_buf, out_buf,
                slice_buf, acc_buf, partials):
    c = lax.axis_index("c")
    s = lax.axis_index("s")
    wid = c * _NS + s

    zeros = jnp.zeros((_L,), jnp.float32)
    ones = jnp.ones((_L,), jnp.float32)

    @plsc.parallel_loop(0, _HIST_PAD, step=_L, unroll=16)
    def zero_hist(i):
        hist[pl.ds(i, _L)] = zeros

    hbase = s * _E_HIST
    pltpu.sync_copy(pdst_hbm.at[pl.ds(hbase, _E_HIST)], idx_buf)

    @plsc.parallel_loop(0, _E_HIST, step=_L, unroll=10)
    def scat_in(i):
        v = idx_buf[pl.ds(i, _L)]
        plsc.addupdate_scatter(hist, [v], ones)

    pltpu.sync_copy(psrc_hbm.at[pl.ds(hbase, _E_HIST)], idx_buf)

    @plsc.parallel_loop(0, _E_HIST, step=_L, unroll=10)
    def scat_out(i):
        v = idx_buf[pl.ds(i, _L)] + _OUT_OFF
        plsc.addupdate_scatter(hist, [v], ones)

    pltpu.sync_copy(hist, partials.at[pl.ds(s * _HIST_PAD, _HIST_PAD)])
    plsc.subcore_barrier()

    @plsc.parallel_loop(0, _SLICE, step=_L, unroll=16)
    def zero_acc(i):
        acc_buf[pl.ds(i, _L)] = zeros

    def red_tile(t, _):
        pltpu.sync_copy(partials.at[pl.ds(t * _HIST_PAD + s * _SLICE, _SLICE)],
                        slice_buf)

        @plsc.parallel_loop(0, _SLICE, step=_L, unroll=16)
        def red_vec(i):
            sl = pl.ds(i, _L)
            acc_buf[sl] = acc_buf[sl] + slice_buf[sl]
        return 0
    lax.fori_loop(0, _NS, red_tile, 0)

    pltpu.sync_copy(acc_buf, partials.at[pl.ds(s * _SLICE, _SLICE)])
    plsc.subcore_barrier()
    pltpu.sync_copy(partials.at[pl.ds(0, _HIST_PAD)], hist)

    gbase = wid * _E_GATH
    pltpu.sync_copy(tail_hbm.at[pl.ds(gbase, _E_GATH)], tail_buf)
    pltpu.sync_copy(head_hbm.at[pl.ds(gbase, _E_GATH)], head_buf)

    @plsc.parallel_loop(0, _E_GATH, step=_L, unroll=10)
    def gath(i):
        sl = pl.ds(i, _L)
        a = plsc.load_gather(hist, [tail_buf[sl]])
        b = plsc.load_gather(hist, [head_buf[sl] + _OUT_OFF])
        out_buf[sl] = (a + b) * jnp.float32(0.5)

    pltpu.sync_copy(out_buf, out_hbm.at[pl.ds(gbase, _E_GATH)])


def kernel(z, edge_index, pos_edge_index):
    del z
    head = edge_index[0]
    tail = edge_index[1]
    psrc = pos_edge_index[0]
    pdst = pos_edge_index[1]
    return _sds_kernel(head, tail, psrc, pdst)


# speedup vs baseline: 1.0757x; 1.0757x over previous
import functools

import jax
import jax.numpy as jnp
from jax import lax
from jax.experimental import pallas as pl
from jax.experimental.pallas import tpu as pltpu
from jax.experimental.pallas import tpu_sc as plsc

_N_NODES = 10000
_N_EDGES = 320000
_L = 16
_NS = 16
_NC = 1
_NW = _NC * _NS
_HIST_PAD = 20480
_OUT_OFF = 10240
_SLICE = _HIST_PAD // _NS
_E_HIST = _N_EDGES // _NS
_E_GATH = _N_EDGES // _NW

_mesh = plsc.VectorSubcoreMesh(core_axis_name="c", subcore_axis_name="s",
                               num_cores=_NC)


@functools.partial(
    pl.kernel,
    mesh=_mesh,
    out_type=jax.ShapeDtypeStruct((_N_EDGES,), jnp.float32),
    scratch_types=[
        pltpu.VMEM((_HIST_PAD,), jnp.float32),
        pltpu.VMEM((_E_HIST // 2,), jnp.int32),
        pltpu.VMEM((_E_HIST // 2,), jnp.int32),
        pltpu.VMEM((_E_GATH,), jnp.int32),
        pltpu.VMEM((_E_GATH,), jnp.int32),
        pltpu.VMEM((_E_GATH,), jnp.float32),
        pltpu.VMEM((_SLICE,), jnp.float32),
        pltpu.VMEM((_SLICE,), jnp.float32),
        pltpu.VMEM_SHARED((_NS * _HIST_PAD,), jnp.float32),
        pltpu.SemaphoreType.DMA,
        pltpu.SemaphoreType.DMA,
        pltpu.SemaphoreType.DMA,
        pltpu.SemaphoreType.DMA,
    ],
    compiler_params=pltpu.CompilerParams(needs_layout_passes=False),
)
def _sds_kernel(head_hbm, tail_hbm, psrc_hbm, pdst_hbm, out_hbm,
                hist, buf_a, buf_b, tail_buf, head_buf, out_buf,
                slice_buf, acc_buf, partials,
                sem_a, sem_b, sem_tail, sem_head):
    s = lax.axis_index("s")

    zeros = jnp.zeros((_L,), jnp.float32)
    ones = jnp.ones((_L,), jnp.float32)

    hbase = s * _E_HIST
    gbase = s * _E_GATH
    cp_tail = pltpu.async_copy(tail_hbm.at[pl.ds(gbase, _E_GATH)], tail_buf,
                               sem_tail)
    cp_head = pltpu.async_copy(head_hbm.at[pl.ds(gbase, _E_GATH)], head_buf,
                               sem_head)

    _HC = _E_HIST // 2
    chunks = [(pdst_hbm, 0, 0), (pdst_hbm, _HC, 0),
              (psrc_hbm, 0, _OUT_OFF), (psrc_hbm, _HC, _OUT_OFF)]
    bufs = [buf_a, buf_b]
    sems = [sem_a, sem_b]

    def start_chunk(k):
        src, off, _ = chunks[k]
        return pltpu.async_copy(src.at[pl.ds(hbase + off, _HC)],
                                bufs[k % 2], sems[k % 2])

    cps = [start_chunk(0), start_chunk(1)]

    @plsc.parallel_loop(0, _HIST_PAD, step=_L, unroll=16)
    def zero_hist(i):
        hist[pl.ds(i, _L)] = zeros

    @plsc.parallel_loop(0, _SLICE, step=_L, unroll=16)
    def zero_acc(i):
        acc_buf[pl.ds(i, _L)] = zeros

    for k in range(4):
        cps[k].wait()
        buf = bufs[k % 2]
        voff = chunks[k][2]

        @plsc.parallel_loop(0, _HC, step=_L, unroll=5)
        def scat(i, buf=buf, voff=voff):
            v = buf[pl.ds(i, _L)] + voff
            plsc.addupdate_scatter(hist, [v], ones)

        if k + 2 < 4:
            cps.append(start_chunk(k + 2))

    pltpu.sync_copy(hist, partials.at[pl.ds(s * _HIST_PAD, _HIST_PAD)])
    plsc.subcore_barrier()

    def red_tile(t, _):
        pltpu.sync_copy(partials.at[pl.ds(t * _HIST_PAD + s * _SLICE, _SLICE)],
                        slice_buf)

        @plsc.parallel_loop(0, _SLICE, step=_L, unroll=16)
        def red_vec(i):
            sl = pl.ds(i, _L)
            acc_buf[sl] = acc_buf[sl] + slice_buf[sl]
        return 0
    lax.fori_loop(0, _NS, red_tile, 0)

    pltpu.sync_copy(acc_buf, partials.at[pl.ds(s * _SLICE, _SLICE)])
    plsc.subcore_barrier()
    pltpu.sync_copy(partials.at[pl.ds(0, _HIST_PAD)], hist)

    cp_tail.wait()
    cp_head.wait()

    @plsc.parallel_loop(0, _E_GATH, step=_L, unroll=10)
    def gath(i):
        sl = pl.ds(i, _L)
        a = plsc.load_gather(hist, [tail_buf[sl]])
        b = plsc.load_gather(hist, [head_buf[sl] + _OUT_OFF])
        out_buf[sl] = (a + b) * jnp.float32(0.5)

    pltpu.sync_copy(out_buf, out_hbm.at[pl.ds(gbase, _E_GATH)])


def kernel(z, edge_index, pos_edge_index):
    del z
    head = edge_index[0]
    tail = edge_index[1]
    psrc = pos_edge_index[0]
    pdst = pos_edge_index[1]
    return _sds_kernel(head, tail, psrc, pdst)
